# baseline (speedup 1.0000x reference)
import functools

import jax
import jax.numpy as jnp
from jax import lax
from jax.experimental import pallas as pl
from jax.experimental.pallas import tpu as pltpu
from jax.experimental.pallas import tpu_sc as plsc

D = 512
B = 4096 * 50
NC = 2
NS = 16
NW = NC * NS
BPW = B // NW
C = 80
NCHUNK = BPW // C
NPAIR = NCHUNK // 2

_mesh = plsc.VectorSubcoreMesh(core_axis_name="c", subcore_axis_name="s")


@functools.partial(
    pl.kernel,
    mesh=_mesh,
    out_type=jax.ShapeDtypeStruct((B, D), jnp.float32),
    scratch_types=[
        pltpu.VMEM((BPW,), jnp.int32),
        pltpu.VMEM((2, C, D), jnp.float32),
        pltpu.SemaphoreType.DMA,
        pltpu.SemaphoreType.DMA,
    ],
)
def _embed_gather(x_hbm, table_hbm, out_hbm, idx_v, rows_v, sem0, sem1):
    wid = lax.axis_index("s") * NC + lax.axis_index("c")
    base = wid * BPW
    pltpu.sync_copy(x_hbm.at[pl.ds(base, BPW)], idx_v)
    sems = (sem0, sem1)

    def gather(c, b):
        pltpu.async_copy(
            table_hbm.at[idx_v.at[pl.ds(c * C, C)]], rows_v.at[b], sems[b]
        )

    def wait_gather(b):
        pltpu.make_async_copy(
            table_hbm.at[pl.ds(0, C)], rows_v.at[b], sems[b]
        ).wait()

    gather(0, 0)
    gather(1, 1)

    def step(i, carry):
        for b in range(2):
            c = 2 * i + b
            wait_gather(b)
            pltpu.sync_copy(rows_v.at[b], out_hbm.at[pl.ds(base + c * C, C)])

            @pl.when(i < NPAIR - 1)
            def _():
                gather(c + 2, b)

        return carry

    lax.fori_loop(0, NPAIR, step, 0)


def kernel(x, table):
    out = _embed_gather(x.reshape(-1), table)
    return out.reshape(x.shape[0], x.shape[1], D)
